# baseline (speedup 1.0000x reference)
import functools

import jax
import jax.numpy as jnp
from jax import lax
from jax.experimental import pallas as pl
from jax.experimental.pallas import tpu as pltpu
from jax.experimental.pallas import tpu_sc as plsc

N_NODES = 10000
D = 128
N_EDGES = 320000

NC = 2
NS = 16
NW = NC * NS
EDGES_PER_TILE = N_EDGES // NW
CHUNK = 128
N_CHUNKS = -(-EDGES_PER_TILE // CHUNK)
PAD_PER_TILE = N_CHUNKS * CHUNK - EDGES_PER_TILE
PAD_BIN = N_NODES
N_PAIRS = (N_CHUNKS - 1) // 2
CHUNK_D = 80
N_CHUNKS_D = EDGES_PER_TILE // CHUNK_D
N_PAD = 10240
ROWS_PER_TILE = N_PAD // NS
OUT_SUB = ROWS_PER_TILE // CHUNK

_mesh = plsc.VectorSubcoreMesh(
    core_axis_name="c", subcore_axis_name="s", num_cores=NC, num_subcores=NS
)


@functools.partial(
    pl.kernel,
    out_type=jax.ShapeDtypeStruct((NC * N_NODES,), jnp.float32),
    mesh=_mesh,
    scratch_types=[
        pltpu.VMEM((N_CHUNKS_D, CHUNK_D), jnp.int32),
        pltpu.VMEM((CHUNK_D,), jnp.float32),
        pltpu.VMEM((1000,), jnp.float32),
        pltpu.VMEM_SHARED((N_NODES,), jnp.float32),
        pltpu.SemaphoreType.DMA,
        pltpu.SemaphoreType.DMA,
    ],
)
def _deg_kernel(dst_hbm, ones_hbm, zeros_hbm, out_hbm, didx_m, ones_v, stage_v,
                hist_s, sem0, sem1):
    c = lax.axis_index("c")
    s = lax.axis_index("s")
    wid = c * NS + s

    @pl.when(s < 10)
    def _():
        pltpu.sync_copy(zeros_hbm, stage_v)
        pltpu.sync_copy(stage_v, hist_s.at[pl.ds(s * 1000, 1000)])

    pltpu.sync_copy(ones_hbm, ones_v)
    pltpu.sync_copy(dst_hbm.at[wid], didx_m)
    plsc.subcore_barrier()

    def body(j, carry):
        pltpu.async_copy(ones_v, hist_s.at[didx_m.at[j]], sem0, add=True)
        pltpu.make_async_copy(ones_v, hist_s.at[didx_m.at[0]], sem0).wait()
        return carry

    lax.fori_loop(0, N_CHUNKS_D, body, 0)
    plsc.subcore_barrier()

    @pl.when(s < 10)
    def _():
        pltpu.sync_copy(hist_s.at[pl.ds(s * 1000, 1000)], stage_v)
        pltpu.sync_copy(stage_v, out_hbm.at[pl.ds(c * N_NODES + s * 1000, 1000)])


@functools.partial(
    pl.kernel,
    out_type=jax.ShapeDtypeStruct((NC, N_PAD, D), jnp.float32),
    mesh=_mesh,
    scratch_types=[
        pltpu.VMEM((CHUNK,), jnp.int32),
        pltpu.VMEM((CHUNK,), jnp.int32),
        pltpu.VMEM((N_CHUNKS, CHUNK), jnp.int32),
        pltpu.VMEM((CHUNK, D), jnp.float32),
        pltpu.VMEM((CHUNK, D), jnp.float32),
        pltpu.VMEM_SHARED((N_PAD, D), jnp.float32),
        pltpu.SemaphoreType.DMA,
        pltpu.SemaphoreType.DMA,
        pltpu.SemaphoreType.DMA,
        pltpu.SemaphoreType.DMA,
        pltpu.SemaphoreType.DMA,
        pltpu.SemaphoreType.DMA,
    ],
)
def _agg_kernel(y_hbm, src_hbm, dst_hbm, zeros_hbm, out_hbm,
                sidx0, sidx1, didx_m, rows0, rows1, hacc_s,
                gsem0, gsem1, ssem0, ssem1, isem0, isem1):
    c = lax.axis_index("c")
    s = lax.axis_index("s")
    wid = c * NS + s
    row0 = s * ROWS_PER_TILE

    pltpu.sync_copy(zeros_hbm, rows0)

    def zbody(i, carry):
        pltpu.sync_copy(rows0, hacc_s.at[pl.ds(row0 + i * CHUNK, CHUNK)])
        return carry

    lax.fori_loop(0, OUT_SUB, zbody, 0)

    pltpu.sync_copy(dst_hbm.at[wid], didx_m)

    ibase = wid * (N_CHUNKS * CHUNK)

    def i_start(j, ib, sem):
        pltpu.async_copy(src_hbm.at[pl.ds(ibase + j * CHUNK, CHUNK)], ib, sem)

    def i_wait(ib, sem):
        pltpu.make_async_copy(src_hbm.at[pl.ds(0, CHUNK)], ib, sem).wait()

    def g_start(ib, buf, sem):
        pltpu.async_copy(y_hbm.at[ib], buf, sem)

    def g_wait(ib, buf, sem):
        pltpu.make_async_copy(y_hbm.at[ib], buf, sem).wait()

    def s_start(j, buf, sem):
        pltpu.async_copy(buf, hacc_s.at[didx_m.at[j]], sem, add=True)

    def s_wait(buf, sem):
        pltpu.make_async_copy(buf, hacc_s.at[didx_m.at[0]], sem).wait()

    i_start(0, sidx0, isem0)
    i_start(1, sidx1, isem1)
    i_wait(sidx0, isem0)
    g_start(sidx0, rows0, gsem0)
    i_wait(sidx1, isem1)
    g_start(sidx1, rows1, gsem1)
    plsc.subcore_barrier()

    def pair(jj, carry):
        j = 2 * jj
        g_wait(sidx0, rows0, gsem0)

        @pl.when(j + 2 < N_CHUNKS)
        def _():
            i_start(j + 2, sidx0, isem0)

        s_start(j, rows0, ssem0)
        s_wait(rows0, ssem0)

        @pl.when(j + 2 < N_CHUNKS)
        def _():
            i_wait(sidx0, isem0)
            g_start(sidx0, rows0, gsem0)

        g_wait(sidx1, rows1, gsem1)

        @pl.when(j + 3 < N_CHUNKS)
        def _():
            i_start(j + 3, sidx1, isem1)

        s_start(j + 1, rows1, ssem1)
        s_wait(rows1, ssem1)

        @pl.when(j + 3 < N_CHUNKS)
        def _():
            i_wait(sidx1, isem1)
            g_start(sidx1, rows1, gsem1)

        return carry

    lax.fori_loop(0, N_PAIRS, pair, 0)
    g_wait(sidx0, rows0, gsem0)
    s_start(N_CHUNKS - 1, rows0, ssem0)
    s_wait(rows0, ssem0)
    plsc.subcore_barrier()

    def obody(i, carry):
        r = row0 + i * CHUNK
        pltpu.sync_copy(hacc_s.at[pl.ds(r, CHUNK)], rows0)
        pltpu.sync_copy(rows0, out_hbm.at[c, pl.ds(r, CHUNK)])
        return carry

    lax.fori_loop(0, OUT_SUB, obody, 0)


def _mm_body(x_ref, w_ref, degt_ref, y_ref, dinv_ref):
    xw = jnp.dot(x_ref[...], w_ref[...], preferred_element_type=jnp.float32)
    deg = degt_ref[:, 0:1] + degt_ref[:, 1:2] + 1.0
    dinv = lax.rsqrt(deg)
    dinv_ref[...] = dinv
    y_ref[...] = xw * dinv


_mm = pl.pallas_call(
    _mm_body,
    out_shape=[
        jax.ShapeDtypeStruct((N_NODES, D), jnp.float32),
        jax.ShapeDtypeStruct((N_NODES, 1), jnp.float32),
    ],
)


def _fin_body(h0_ref, h1_ref, y_ref, dinv_ref, x_ref, b_ref, g_ref, be_ref, o_ref):
    h = (h0_ref[...] + h1_ref[...] + y_ref[...]) * dinv_ref[...] + b_ref[...]
    h = jnp.maximum(h, 0.0) + x_ref[...]
    m = jnp.mean(h, axis=0, keepdims=True)
    d = h - m
    v = jnp.mean(d * d, axis=0, keepdims=True)
    o_ref[...] = d * lax.rsqrt(v + 1e-5) * g_ref[...] + be_ref[...]


_fin = pl.pallas_call(
    _fin_body,
    out_shape=jax.ShapeDtypeStruct((N_NODES, D), jnp.float32),
)


def kernel(x, edge_index, W, b, gamma, beta):
    ei = edge_index.astype(jnp.int32)
    src2 = ei[0].reshape(NW, EDGES_PER_TILE)
    dst2 = ei[1].reshape(NW, EDGES_PER_TILE)
    src3 = jnp.pad(src2, ((0, 0), (0, PAD_PER_TILE))).reshape(-1)
    dst3 = jnp.pad(dst2, ((0, 0), (0, PAD_PER_TILE)),
                   constant_values=PAD_BIN).reshape(NW, N_CHUNKS, CHUNK)
    dst3d = dst2.reshape(NW, N_CHUNKS_D, CHUNK_D)
    ones_c = jnp.ones((CHUNK_D,), jnp.float32)
    zeros_n = jnp.zeros((1000,), jnp.float32)
    deg_part = _deg_kernel(dst3d, ones_c, zeros_n)
    degt = deg_part.reshape(NC, N_NODES).T
    y, dinv = _mm(x, W, degt)
    zeros_rows = jnp.zeros((CHUNK, D), jnp.float32)
    h_part = _agg_kernel(y, src3, dst3, zeros_rows)
    return _fin(h_part[0, :N_NODES], h_part[1, :N_NODES], y, dinv, x,
                b.reshape(1, D), gamma.reshape(1, D), beta.reshape(1, D))


# speedup vs baseline: 1.5887x; 1.5887x over previous
import functools

import jax
import jax.numpy as jnp
from jax import lax
from jax.experimental import pallas as pl
from jax.experimental.pallas import tpu as pltpu
from jax.experimental.pallas import tpu_sc as plsc

N_NODES = 10000
D = 128
N_EDGES = 320000

NC = 2
NS = 16
NW = NC * NS
EDGES_PER_TILE = N_EDGES // NW
CHUNK = 80
N_CHUNKS = EDGES_PER_TILE // CHUNK
N_TRI = (N_CHUNKS - 2) // 3
N_PAD = 10240
ROWS_PER_TILE = N_PAD // NS
OUT_SUB = ROWS_PER_TILE // CHUNK

_mesh = plsc.VectorSubcoreMesh(
    core_axis_name="c", subcore_axis_name="s", num_cores=NC, num_subcores=NS
)


@functools.partial(
    pl.kernel,
    out_type=jax.ShapeDtypeStruct((NC * N_NODES,), jnp.float32),
    mesh=_mesh,
    scratch_types=[
        pltpu.VMEM((N_CHUNKS, CHUNK), jnp.int32),
        pltpu.VMEM((CHUNK,), jnp.float32),
        pltpu.VMEM((1000,), jnp.float32),
        pltpu.VMEM_SHARED((N_NODES,), jnp.float32),
        pltpu.SemaphoreType.DMA,
    ],
)
def _deg_kernel(dst_hbm, ones_hbm, zeros_hbm, out_hbm, didx_m, ones_v, stage_v,
                hist_s, sem0):
    c = lax.axis_index("c")
    s = lax.axis_index("s")
    wid = c * NS + s

    @pl.when(s < 10)
    def _():
        pltpu.sync_copy(zeros_hbm, stage_v)
        pltpu.sync_copy(stage_v, hist_s.at[pl.ds(s * 1000, 1000)])

    pltpu.sync_copy(ones_hbm, ones_v)
    pltpu.sync_copy(dst_hbm.at[wid], didx_m)
    plsc.subcore_barrier()

    def body(j, carry):
        pltpu.async_copy(ones_v, hist_s.at[didx_m.at[j]], sem0, add=True)
        pltpu.make_async_copy(ones_v, hist_s.at[didx_m.at[0]], sem0).wait()
        return carry

    lax.fori_loop(0, N_CHUNKS, body, 0)
    plsc.subcore_barrier()

    @pl.when(s < 10)
    def _():
        pltpu.sync_copy(hist_s.at[pl.ds(s * 1000, 1000)], stage_v)
        pltpu.sync_copy(stage_v, out_hbm.at[pl.ds(c * N_NODES + s * 1000, 1000)])


@functools.partial(
    pl.kernel,
    out_type=jax.ShapeDtypeStruct((NC, N_PAD, D), jnp.float32),
    mesh=_mesh,
    scratch_types=[
        pltpu.VMEM((CHUNK,), jnp.int32),
        pltpu.VMEM((CHUNK,), jnp.int32),
        pltpu.VMEM((CHUNK,), jnp.int32),
        pltpu.VMEM((N_CHUNKS, CHUNK), jnp.int32),
        pltpu.VMEM((CHUNK, D), jnp.float32),
        pltpu.VMEM((CHUNK, D), jnp.float32),
        pltpu.VMEM((CHUNK, D), jnp.float32),
        pltpu.VMEM_SHARED((N_PAD, D), jnp.float32),
        pltpu.SemaphoreType.DMA,
        pltpu.SemaphoreType.DMA,
        pltpu.SemaphoreType.DMA,
        pltpu.SemaphoreType.DMA,
        pltpu.SemaphoreType.DMA,
        pltpu.SemaphoreType.DMA,
        pltpu.SemaphoreType.DMA,
    ],
)
def _agg_kernel(y_hbm, src_hbm, dst_hbm, zeros_hbm, out_hbm,
                ib0, ib1, ib2, didx_m, rows0, rows1, rows2, hacc_s,
                gsem0, gsem1, gsem2, isem0, isem1, isem2, ssem):
    c = lax.axis_index("c")
    s = lax.axis_index("s")
    wid = c * NS + s
    row0 = s * ROWS_PER_TILE
    ibs = (ib0, ib1, ib2)
    rows = (rows0, rows1, rows2)
    gsems = (gsem0, gsem1, gsem2)
    isems = (isem0, isem1, isem2)

    pltpu.sync_copy(zeros_hbm, rows0)

    def zbody(i, carry):
        pltpu.sync_copy(rows0, hacc_s.at[pl.ds(row0 + i * CHUNK, CHUNK)])
        return carry

    lax.fori_loop(0, OUT_SUB, zbody, 0)

    pltpu.sync_copy(dst_hbm.at[wid], didx_m)

    ibase = wid * EDGES_PER_TILE

    def i_start(j, ib, sem):
        pltpu.async_copy(src_hbm.at[pl.ds(ibase + j * CHUNK, CHUNK)], ib, sem)

    def i_wait(ib, sem):
        pltpu.make_async_copy(src_hbm.at[pl.ds(0, CHUNK)], ib, sem).wait()

    def g_start(ib, buf, sem):
        pltpu.async_copy(y_hbm.at[ib], buf, sem)

    def g_wait(ib, buf, sem):
        pltpu.make_async_copy(y_hbm.at[ib], buf, sem).wait()

    def s_sync(j, buf):
        pltpu.async_copy(buf, hacc_s.at[didx_m.at[j]], ssem, add=True)
        pltpu.make_async_copy(buf, hacc_s.at[didx_m.at[0]], ssem).wait()

    for p in range(3):
        i_start(p, ibs[p], isems[p])
    for p in range(2):
        i_wait(ibs[p], isems[p])
        g_start(ibs[p], rows[p], gsems[p])
    plsc.subcore_barrier()

    def tri(jj, carry):
        for p in range(3):
            j = 3 * jj + p
            q = (p + 2) % 3
            g_wait(ibs[p], rows[p], gsems[p])

            @pl.when(j + 3 < N_CHUNKS)
            def _():
                i_start(j + 3, ibs[p], isems[p])

            s_sync(j, rows[p])

            @pl.when(j + 2 < N_CHUNKS)
            def _():
                i_wait(ibs[q], isems[q])
                g_start(ibs[q], rows[q], gsems[q])

        return carry

    lax.fori_loop(0, N_TRI, tri, 0)
    g_wait(ibs[0], rows[0], gsems[0])
    s_sync(N_CHUNKS - 2, rows[0])
    g_wait(ibs[1], rows[1], gsems[1])
    s_sync(N_CHUNKS - 1, rows[1])
    plsc.subcore_barrier()

    def obody(i, carry):
        r = row0 + i * CHUNK
        pltpu.sync_copy(hacc_s.at[pl.ds(r, CHUNK)], rows0)
        pltpu.sync_copy(rows0, out_hbm.at[c, pl.ds(r, CHUNK)])
        return carry

    lax.fori_loop(0, OUT_SUB, obody, 0)


def _mm_body(x_ref, w_ref, degt_ref, y_ref, dinv_ref):
    xw = jnp.dot(x_ref[...], w_ref[...], preferred_element_type=jnp.float32)
    deg = degt_ref[:, 0:1] + degt_ref[:, 1:2] + 1.0
    dinv = lax.rsqrt(deg)
    dinv_ref[...] = dinv
    y_ref[...] = xw * dinv


_mm = pl.pallas_call(
    _mm_body,
    out_shape=[
        jax.ShapeDtypeStruct((N_NODES, D), jnp.float32),
        jax.ShapeDtypeStruct((N_NODES, 1), jnp.float32),
    ],
)


def _fin_body(h0_ref, h1_ref, y_ref, dinv_ref, x_ref, b_ref, g_ref, be_ref, o_ref):
    h = (h0_ref[...] + h1_ref[...] + y_ref[...]) * dinv_ref[...] + b_ref[...]
    h = jnp.maximum(h, 0.0) + x_ref[...]
    m = jnp.mean(h, axis=0, keepdims=True)
    d = h - m
    v = jnp.mean(d * d, axis=0, keepdims=True)
    o_ref[...] = d * lax.rsqrt(v + 1e-5) * g_ref[...] + be_ref[...]


_fin = pl.pallas_call(
    _fin_body,
    out_shape=jax.ShapeDtypeStruct((N_NODES, D), jnp.float32),
)


def kernel(x, edge_index, W, b, gamma, beta):
    ei = edge_index.astype(jnp.int32)
    srcf = ei[0]
    dst3 = ei[1].reshape(NW, N_CHUNKS, CHUNK)
    ones_c = jnp.ones((CHUNK,), jnp.float32)
    zeros_n = jnp.zeros((1000,), jnp.float32)
    deg_part = _deg_kernel(dst3, ones_c, zeros_n)
    degt = deg_part.reshape(NC, N_NODES).T
    y, dinv = _mm(x, W, degt)
    zeros_rows = jnp.zeros((CHUNK, D), jnp.float32)
    h_part = _agg_kernel(y, srcf, dst3, zeros_rows)
    return _fin(h_part[0, :N_NODES], h_part[1, :N_NODES], y, dinv, x,
                b.reshape(1, D), gamma.reshape(1, D), beta.reshape(1, D))
